# Initial kernel scaffold; baseline (speedup 1.0000x reference)
import jax
import jax.numpy as jnp
from jax.experimental import pallas as pl


def kernel(x, edge_index, edge_attr, batchSize, params):
    raise NotImplementedError("write your pallas kernel here")


# speedup vs baseline: 2.9361x; 2.9361x over previous
import jax
import jax.numpy as jnp
from jax.experimental import pallas as pl

F32 = jnp.float32
_N = 200
_E = 6400
_D = 768
_NH = 12
_DH = 64
_FF = 2048
_C1 = 1152
_RPT = 10
_TILE = _RPT * _N
_NT = _N // _RPT
_HALO = 7
_TOT = _N * _N


def _gelu(x):
    return 0.5 * x * (1.0 + jax.lax.erf(x * 0.7071067811865476))


def _mm(a, b):
    return jnp.dot(a, b, preferred_element_type=F32)


def _gcn_body(ei_ref, ew_ref, x_ref, w1_ref, b1_ref, w2_ref, b2_ref,
              w3_ref, b3_ref, out_ref):
    src = ei_ref[0:1, :]
    dst = ei_ref[1:2, :]
    ew = ew_ref[0:1, :]
    niota = jax.lax.broadcasted_iota(jnp.int32, (_N, _E), 0)
    md = (niota == dst).astype(F32)
    ms = (niota == src).astype(F32)
    mdw = md * ew
    a = jax.lax.dot_general(mdw, ms, (((1,), (1,)), ((), ())),
                            preferred_element_type=F32)
    deg = jnp.sum(mdw, axis=1, keepdims=True) + 1.0
    dinv = jax.lax.rsqrt(deg)
    eye = (jax.lax.broadcasted_iota(jnp.int32, (_N, _N), 0)
           == jax.lax.broadcasted_iota(jnp.int32, (_N, _N), 1)).astype(F32)
    an = dinv * (a + eye) * dinv.reshape(1, _N)
    h1 = _gelu(_mm(an, _mm(x_ref[...], w1_ref[...])) + b1_ref[...])
    h2 = _gelu(_mm(an, _mm(h1, w2_ref[...])) + b2_ref[...])
    out_ref[...] = _mm(an, _mm(h2, w3_ref[...])) + b3_ref[...]


def _ln(x, g, b):
    m = jnp.mean(x, axis=-1, keepdims=True)
    v = jnp.mean((x - m) ** 2, axis=-1, keepdims=True)
    return (x - m) * jax.lax.rsqrt(v + 1e-5) * g + b


def _tf_body(x_ref, wq_ref, bq_ref, wk_ref, bk_ref, wv_ref, bv_ref,
             wo_ref, bo_ref, w1_ref, b1_ref, w2_ref, b2_ref,
             g1_ref, be1_ref, g2_ref, be2_ref, out_ref):
    l = pl.program_id(0)
    xin = jnp.where(l == 0, x_ref[...], out_ref[...])
    q = _mm(xin, wq_ref[0]) + bq_ref[0]
    k = _mm(xin, wk_ref[0]) + bk_ref[0]
    v = _mm(xin, wv_ref[0]) + bv_ref[0]
    heads = []
    for hd in range(_NH):
        sl = slice(hd * _DH, (hd + 1) * _DH)
        s = jax.lax.dot_general(q[:, sl], k[:, sl], (((1,), (1,)), ((), ())),
                                preferred_element_type=F32) * (1.0 / 8.0)
        p = jax.nn.softmax(s, axis=-1)
        heads.append(_mm(p, v[:, sl]))
    o = _mm(jnp.concatenate(heads, axis=1), wo_ref[0]) + bo_ref[0]
    x1 = _ln(xin + o, g1_ref[0], be1_ref[0])
    f = _mm(jax.nn.relu(_mm(x1, w1_ref[0]) + b1_ref[0]), w2_ref[0]) + b2_ref[0]
    out_ref[...] = _gelu(_ln(x1 + f, g2_ref[0], be2_ref[0]))


def _prep_body(h_ref, wo0, wo1, wo2, wd0, wd1, wd2, b1, wct, wcb, bc,
               u_ref, v_ref, c0_ref, c199_ref, uc_ref, vc_ref):
    h = h_ref[...]
    hp = jnp.concatenate([h[_N - 1:], h[:_N - 1]], 0)
    hn = jnp.concatenate([h[1:], h[:1]], 0)
    u_ref[...] = _mm(h, wo0[...] + wo1[...] + wo2[...]) + b1[...]
    v_ref[...] = _mm(hp, wd0[...]) + _mm(h, wd1[...]) + _mm(hn, wd2[...])
    r0 = _mm(hp - h, wo0[...])
    row0 = -_mm(h[0:1], wo0[...]) - _mm(h[_N - 1:], wd0[...])
    c0_ref[...] = jnp.concatenate([row0, r0[1:]], 0)
    r1 = _mm(hn - h, wo2[...])
    row199 = -_mm(h[_N - 1:], wo2[...]) - _mm(h[0:1], wd2[...])
    c199_ref[...] = jnp.concatenate([r1[:_N - 1], row199], 0)
    uc_ref[...] = _mm(h, wct[...]) + bc[...]
    vc_ref[...] = _mm(h, wcb[...])


def _pair_body(*refs):
    (u_ref, v_ref, c0_ref, c199_ref, uc_ref, vc_ref,
     up_ref, un_ref, c199p_ref, c0n_ref) = refs[0:10]
    conv = refs[10:10 + 28]
    cls = refs[38:44]
    c_out, z_out = refs[44], refs[45]

    t = pl.program_id(0)
    g0 = t * _TILE - _HALO

    vfull = v_ref[...]
    jiota = jax.lax.broadcasted_iota(jnp.int32, (1, _N, 1), 1)
    zm = u_ref[0][:, None, :] + vfull[None, :, :]
    zm = zm + jnp.where(jiota == 0, c0_ref[0][:, None, :], 0.0)
    zm = zm + jnp.where(jiota == _N - 1, c199_ref[0][:, None, :], 0.0)
    zm = _gelu(zm).reshape(_TILE, _C1)

    hiota = jax.lax.broadcasted_iota(jnp.int32, (_HALO, 1), 0)
    zl = up_ref[0] + vfull[_N - _HALO:_N, :]
    zl = zl + jnp.where(hiota == _HALO - 1, c199p_ref[0], 0.0)
    zl = _gelu(zl)
    zr = un_ref[0] + vfull[0:_HALO, :]
    zr = zr + jnp.where(hiota == 0, c0n_ref[0], 0.0)
    zr = _gelu(zr)

    z = jnp.concatenate([zl, zm, zr], 0)
    off = 0

    def mask(y, off):
        ln = y.shape[0]
        pos = g0 + off + jax.lax.broadcasted_iota(jnp.int32, (ln, 1), 0)
        return jnp.where((pos >= 0) & (pos < _TOT), y, 0.0)

    z = mask(z, 0)
    for li in range(7):
        a0, a1, a2, bb = conv[4 * li:4 * li + 4]
        ln = z.shape[0]
        y = (_mm(z[0:ln - 2], a0[...]) + _mm(z[1:ln - 1], a1[...])
             + _mm(z[2:ln], a2[...]) + bb[...])
        if li < 6:
            y = _gelu(y)
        off += 1
        z = mask(y, off) if li < 6 else y
    z_out[...] = z

    w2, b2, w3, b3, w4, b4 = cls
    c1 = jnp.tanh(uc_ref[0][:, None, :]
                  + vc_ref[...][None, :, :]).reshape(_TILE, _D)
    c2 = jax.nn.relu(_mm(c1, w2[...]) + b2[...])
    c3 = jnp.tanh(_mm(c2, w3[...]) + b3[...])
    c_out[...] = _mm(c3, w4[...]) + b4[...]


def kernel(x, edge_index, edge_attr, batchSize, params):
    gp = params['gcn']
    g = pl.pallas_call(
        _gcn_body,
        out_shape=jax.ShapeDtypeStruct((_N, _D), F32),
    )(edge_index.astype(jnp.int32), edge_attr.reshape(1, _E).astype(F32),
      x, gp[0]['W'], gp[0]['b'].reshape(1, -1),
      gp[1]['W'], gp[1]['b'].reshape(1, -1),
      gp[2]['W'], gp[2]['b'].reshape(1, -1))

    h = g * jnp.asarray(batchSize, F32)

    tf = params['tf']
    def stk(name):
        return jnp.stack([p[name] for p in tf])
    def stkb(name):
        return jnp.stack([p[name] for p in tf]).reshape(12, 1, -1)
    mat = lambda: pl.BlockSpec((1, _D, _D), lambda l: (l, 0, 0))
    vec = lambda: pl.BlockSpec((1, 1, _D), lambda l: (l, 0, 0))
    h = pl.pallas_call(
        _tf_body,
        grid=(12,),
        in_specs=[pl.BlockSpec((_N, _D), lambda l: (0, 0)),
                  mat(), vec(), mat(), vec(), mat(), vec(), mat(), vec(),
                  pl.BlockSpec((1, _D, _FF), lambda l: (l, 0, 0)),
                  pl.BlockSpec((1, 1, _FF), lambda l: (l, 0, 0)),
                  pl.BlockSpec((1, _FF, _D), lambda l: (l, 0, 0)),
                  vec(), vec(), vec(), vec(), vec()],
        out_specs=pl.BlockSpec((_N, _D), lambda l: (0, 0)),
        out_shape=jax.ShapeDtypeStruct((_N, _D), F32),
    )(h, stk('Wq'), stkb('bq'), stk('Wk'), stkb('bk'),
      stk('Wv'), stkb('bv'), stk('Wo'), stkb('bo'),
      stk('W1'), stkb('b1'), stk('W2'), stkb('b2'),
      stkb('g1'), stkb('be1'), stkb('g2'), stkb('be2'))

    cw1 = params['conv'][0]['W']
    wo = [cw1[:, :_D, k].T for k in range(3)]
    wd = [cw1[:, _D:, k].T for k in range(3)]
    cb1 = params['conv'][0]['b'].reshape(1, -1)
    clw1 = params['cls'][0]['W']
    tabs = pl.pallas_call(
        _prep_body,
        out_shape=[jax.ShapeDtypeStruct((_N, _C1), F32)] * 4
                  + [jax.ShapeDtypeStruct((_N, _D), F32)] * 2,
    )(h, wo[0], wo[1], wo[2], wd[0], wd[1], wd[2], cb1,
      clw1[:_D], clw1[_D:], params['cls'][0]['b'].reshape(1, -1))
    u_t, v_t, c0_t, c199_t, uc_t, vc_t = tabs

    conv_args = []
    for i in range(1, 8):
        w = params['conv'][i]['W']
        b = params['conv'][i]['b']
        if i == 7:
            w = jnp.pad(w, ((0, 7), (0, 0), (0, 0)))
            b = jnp.pad(b, (0, 7))
        for k in range(3):
            conv_args.append(w[:, :, k].T)
        conv_args.append(b.reshape(1, -1))

    cp = params['cls']
    cls_args = [cp[1]['W'], cp[1]['b'].reshape(1, -1),
                cp[2]['W'], cp[2]['b'].reshape(1, -1),
                cp[3]['W'], cp[3]['b'].reshape(1, -1)]

    u3 = u_t.reshape(_NT, _RPT, _C1)
    c03 = c0_t.reshape(_NT, _RPT, _C1)
    c1993 = c199_t.reshape(_NT, _RPT, _C1)
    uc3 = uc_t.reshape(_NT, _RPT, _D)
    idxp = jnp.maximum(jnp.arange(_NT) * _RPT - 1, 0)
    idxn = jnp.minimum(jnp.arange(_NT) * _RPT + _RPT, _N - 1)
    up3 = u_t[idxp][:, None, :]
    un3 = u_t[idxn][:, None, :]
    c199p3 = c199_t[idxp][:, None, :]
    c0n3 = c0_t[idxn][:, None, :]

    tile3 = lambda a: pl.BlockSpec((1,) + a.shape[1:], lambda t: (t, 0, 0))
    const2 = lambda a: pl.BlockSpec(a.shape, lambda t: (0, 0))
    tab_args = [u3, v_t, c03, c1993, uc3, vc_t, up3, un3, c199p3, c0n3]
    tab_spec = [tile3(u3), const2(v_t), tile3(c03), tile3(c1993),
                tile3(uc3), const2(vc_t), tile3(up3), tile3(un3),
                tile3(c199p3), tile3(c0n3)]
    w_spec = [const2(a) for a in conv_args + cls_args]
    c_out, z_out = pl.pallas_call(
        _pair_body,
        grid=(_NT,),
        in_specs=tab_spec + w_spec,
        out_specs=[pl.BlockSpec((_TILE, 8), lambda t: (t, 0)),
                   pl.BlockSpec((_TILE, 8), lambda t: (t, 0))],
        out_shape=[jax.ShapeDtypeStruct((_TOT, 8), F32),
                   jax.ShapeDtypeStruct((_TOT, 8), F32)],
    )(*tab_args, *conv_args, *cls_args)

    return (z_out[:, 0], c_out)


# speedup vs baseline: 2.9852x; 1.0167x over previous
import jax
import jax.numpy as jnp
from jax.experimental import pallas as pl

F32 = jnp.float32
BF16 = jnp.bfloat16
_N = 200
_E = 6400
_D = 768
_NH = 12
_DH = 64
_FF = 2048
_C1 = 1152
_RPT = 10
_TILE = _RPT * _N
_NT = _N // _RPT
_HALO = 7
_TOT = _N * _N

_SQ2 = 1.4142135623730951


def _gelu(x):
    return 0.5 * x * (1.0 + jax.lax.erf(x / _SQ2))


def _d1(a, b, dims=((1,), (0,))):
    return jax.lax.dot_general(a, b, (dims, ((), ())),
                               preferred_element_type=F32)


def _mmx(a, b, dims=((1,), (0,))):
    return _d1(a.astype(BF16), b.astype(BF16), dims)


def _split(a):
    hi = a.astype(BF16)
    return hi, (a - hi.astype(F32)).astype(BF16)


def _dot3(ah, al, bh, bl, dims=((1,), (0,))):
    return _d1(ah, bh, dims) + _d1(al, bh, dims) + _d1(ah, bl, dims)


def _mm3(a, b, dims=((1,), (0,))):
    ah, al = _split(a)
    bh, bl = _split(b)
    return _dot3(ah, al, bh, bl, dims)


def _gcn_body(ei_ref, ew_ref, x_ref, w1_ref, b1_ref, w2_ref, b2_ref,
              w3_ref, b3_ref, out_ref):
    src = ei_ref[0:1, :]
    dst = ei_ref[1:2, :]
    ew = ew_ref[0:1, :]
    niota = jax.lax.broadcasted_iota(jnp.int32, (_N, _E), 0)
    md = (niota == dst).astype(F32)
    ms = (niota == src).astype(F32)
    mdw = md * ew
    a = _mm3(mdw, ms, ((1,), (1,)))
    deg = jnp.sum(mdw, axis=1, keepdims=True) + 1.0
    dinv = 1.0 / jnp.sqrt(deg)
    eye = (jax.lax.broadcasted_iota(jnp.int32, (_N, _N), 0)
           == jax.lax.broadcasted_iota(jnp.int32, (_N, _N), 1)).astype(F32)
    an = dinv * (a + eye) * dinv.reshape(1, _N)
    anh, anl = _split(an)
    def agg(h):
        hh, hl = _split(h)
        return _dot3(anh, anl, hh, hl)
    h1 = _gelu(agg(_mmx(x_ref[...], w1_ref[...])) + b1_ref[...])
    h2 = _gelu(agg(_mmx(h1, w2_ref[...])) + b2_ref[...])
    out_ref[...] = agg(_mmx(h2, w3_ref[...])) + b3_ref[...]


def _ln(x, g, b):
    m = jnp.mean(x, axis=-1, keepdims=True)
    v = jnp.mean((x - m) ** 2, axis=-1, keepdims=True)
    return (x - m) / jnp.sqrt(v + 1e-5) * g + b


def _tf_body(x_ref, wq_ref, bq_ref, wk_ref, bk_ref, wv_ref, bv_ref,
             wo_ref, bo_ref, w1_ref, b1_ref, w2_ref, b2_ref,
             g1_ref, be1_ref, g2_ref, be2_ref, out_ref):
    xin = x_ref[...]
    q = _mmx(xin, wq_ref[...]) + bq_ref[...]
    k = _mmx(xin, wk_ref[...]) + bk_ref[...]
    v = _mmx(xin, wv_ref[...]) + bv_ref[...]
    heads = []
    for hd in range(_NH):
        sl = slice(hd * _DH, (hd + 1) * _DH)
        s = _mmx(q[:, sl], k[:, sl], ((1,), (1,))) / 8.0
        p = jax.nn.softmax(s, axis=-1)
        heads.append(_mmx(p, v[:, sl]))
    o = _mmx(jnp.concatenate(heads, axis=1), wo_ref[...]) + bo_ref[...]
    x1 = _ln(xin + o, g1_ref[...], be1_ref[...])
    f = _mmx(jax.nn.relu(_mmx(x1, w1_ref[...]) + b1_ref[...]),
             w2_ref[...]) + b2_ref[...]
    out_ref[...] = _gelu(_ln(x1 + f, g2_ref[...], be2_ref[...]))


def _prep_body(h_ref, wo0, wo1, wo2, wd0, wd1, wd2, b1, wct, wcb, bc,
               u_ref, v_ref, c0_ref, c199_ref, uc_ref, vc_ref):
    hb = h_ref[...].astype(BF16)
    hp = jnp.concatenate([hb[_N - 1:], hb[:_N - 1]], 0)
    hn = jnp.concatenate([hb[1:], hb[:1]], 0)
    d = lambda a, b: _d1(a, b[...])
    u0 = d(hb, wo0)
    u2 = d(hb, wo2)
    u_ref[...] = (u0 + d(hb, wo1)) + u2 + b1[...]
    v_ref[...] = d(hp, wd0) + d(hb, wd1) + d(hn, wd2)
    c0full = d(hp, wo0) - u0
    row0 = -d(hb[0:1], wo0) - d(hb[_N - 1:], wd0)
    c0_ref[...] = jnp.concatenate([row0, c0full[1:]], 0)
    c199full = d(hn, wo2) - u2
    row199 = -d(hb[_N - 1:], wo2) - d(hb[0:1], wd2)
    c199_ref[...] = jnp.concatenate([c199full[:_N - 1], row199], 0)
    uc_ref[...] = d(hb, wct) + bc[...]
    vc_ref[...] = d(hb, wcb)


def _pair_body(*refs):
    (u_ref, v_ref, c0_ref, c199_ref, uc_ref, vc_ref,
     up_ref, un_ref, c199p_ref, c0n_ref) = refs[0:10]
    conv = refs[10:10 + 28]
    cls = refs[38:44]
    c_out, z_out = refs[44], refs[45]

    t = pl.program_id(0)
    g0 = t * _TILE - _HALO

    vfull = v_ref[...]
    jiota = jax.lax.broadcasted_iota(jnp.int32, (_N, 1), 0)
    m0 = (jiota == 0).astype(F32)
    m199 = (jiota == _N - 1).astype(F32)
    um, c0m, c199m = u_ref[0], c0_ref[0], c199_ref[0]
    rows = []
    for r in range(_RPT):
        blk = vfull + um[r:r + 1, :]
        blk = blk + m0 * c0m[r:r + 1, :] + m199 * c199m[r:r + 1, :]
        rows.append(blk)
    zm = _gelu(jnp.concatenate(rows, 0))

    hiota = jax.lax.broadcasted_iota(jnp.int32, (_HALO, 1), 0)
    zl = up_ref[0] + vfull[_N - _HALO:_N, :]
    zl = zl + jnp.where(hiota == _HALO - 1, c199p_ref[0], 0.0)
    zl = _gelu(zl)
    zr = un_ref[0] + vfull[0:_HALO, :]
    zr = zr + jnp.where(hiota == 0, c0n_ref[0], 0.0)
    zr = _gelu(zr)

    z = jnp.concatenate([zl, zm, zr], 0)
    off = 0

    def mask(y, off):
        ln = y.shape[0]
        pos = g0 + off + jax.lax.broadcasted_iota(jnp.int32, (ln, 1), 0)
        return jnp.where((pos >= 0) & (pos < _TOT), y, 0.0)

    z = mask(z, 0)
    for li in range(7):
        a0, a1, a2, bb = conv[4 * li:4 * li + 4]
        ln = z.shape[0]
        zb = z.astype(BF16)
        y = (_d1(zb[0:ln - 2], a0[...]) + _d1(zb[1:ln - 1], a1[...])
             + _d1(zb[2:ln], a2[...]) + bb[...])
        if li < 6:
            y = _gelu(y)
        off += 1
        z = mask(y, off) if li < 6 else y
    z_out[...] = z

    w2, b2, w3, b3, w4, b4 = cls
    vcf = vc_ref[...]
    ucm = uc_ref[0]
    c1 = jnp.tanh(jnp.concatenate(
        [vcf + ucm[r:r + 1, :] for r in range(_RPT)], 0))
    c2 = jax.nn.relu(_d1(c1.astype(BF16), w2[...]) + b2[...])
    c3 = jnp.tanh(_d1(c2.astype(BF16), w3[...]) + b3[...])
    c_out[...] = _d1(c3.astype(BF16), w4[...]) + b4[...]


def kernel(x, edge_index, edge_attr, batchSize, params):
    gp = params['gcn']
    g = pl.pallas_call(
        _gcn_body,
        out_shape=jax.ShapeDtypeStruct((_N, _D), F32),
    )(edge_index.astype(jnp.int32), edge_attr.reshape(1, _E).astype(F32),
      x, gp[0]['W'], gp[0]['b'].reshape(1, -1),
      gp[1]['W'], gp[1]['b'].reshape(1, -1),
      gp[2]['W'], gp[2]['b'].reshape(1, -1))

    h = g * jnp.asarray(batchSize, F32)

    for p in params['tf']:
        h = pl.pallas_call(
            _tf_body,
            out_shape=jax.ShapeDtypeStruct((_N, _D), F32),
        )(h, p['Wq'], p['bq'].reshape(1, -1), p['Wk'], p['bk'].reshape(1, -1),
          p['Wv'], p['bv'].reshape(1, -1), p['Wo'], p['bo'].reshape(1, -1),
          p['W1'], p['b1'].reshape(1, -1), p['W2'], p['b2'].reshape(1, -1),
          p['g1'].reshape(1, -1), p['be1'].reshape(1, -1),
          p['g2'].reshape(1, -1), p['be2'].reshape(1, -1))

    cw1 = params['conv'][0]['W']
    wo = [cw1[:, :_D, k].T.astype(BF16) for k in range(3)]
    wd = [cw1[:, _D:, k].T.astype(BF16) for k in range(3)]
    cb1 = params['conv'][0]['b'].reshape(1, -1)
    clw1 = params['cls'][0]['W']
    tabs = pl.pallas_call(
        _prep_body,
        out_shape=[jax.ShapeDtypeStruct((_N, _C1), F32)] * 4
                  + [jax.ShapeDtypeStruct((_N, _D), F32)] * 2,
    )(h, wo[0], wo[1], wo[2], wd[0], wd[1], wd[2], cb1,
      clw1[:_D].astype(BF16), clw1[_D:].astype(BF16),
      params['cls'][0]['b'].reshape(1, -1))
    u_t, v_t, c0_t, c199_t, uc_t, vc_t = tabs

    conv_args = []
    for i in range(1, 8):
        w = params['conv'][i]['W']
        b = params['conv'][i]['b']
        if i == 7:
            w = jnp.pad(w, ((0, 7), (0, 0), (0, 0)))
            b = jnp.pad(b, (0, 7))
        for k in range(3):
            conv_args.append(w[:, :, k].T.astype(BF16))
        conv_args.append(b.reshape(1, -1))

    cp = params['cls']
    cls_args = [cp[1]['W'].astype(BF16), cp[1]['b'].reshape(1, -1),
                cp[2]['W'].astype(BF16), cp[2]['b'].reshape(1, -1),
                cp[3]['W'].astype(BF16), cp[3]['b'].reshape(1, -1)]

    u3 = u_t.reshape(_NT, _RPT, _C1)
    c03 = c0_t.reshape(_NT, _RPT, _C1)
    c1993 = c199_t.reshape(_NT, _RPT, _C1)
    uc3 = uc_t.reshape(_NT, _RPT, _D)
    idxp = jnp.maximum(jnp.arange(_NT) * _RPT - 1, 0)
    idxn = jnp.minimum(jnp.arange(_NT) * _RPT + _RPT, _N - 1)
    up3 = u_t[idxp][:, None, :]
    un3 = u_t[idxn][:, None, :]
    c199p3 = c199_t[idxp][:, None, :]
    c0n3 = c0_t[idxn][:, None, :]

    tile3 = lambda a: pl.BlockSpec((1,) + a.shape[1:], lambda t: (t, 0, 0))
    const2 = lambda a: pl.BlockSpec(a.shape, lambda t: (0, 0))
    tab_args = [u3, v_t, c03, c1993, uc3, vc_t, up3, un3, c199p3, c0n3]
    tab_spec = [tile3(u3), const2(v_t), tile3(c03), tile3(c1993),
                tile3(uc3), const2(vc_t), tile3(up3), tile3(un3),
                tile3(c199p3), tile3(c0n3)]
    w_spec = [const2(a) for a in conv_args + cls_args]
    c_out, z_out = pl.pallas_call(
        _pair_body,
        grid=(_NT,),
        in_specs=tab_spec + w_spec,
        out_specs=[pl.BlockSpec((_TILE, 8), lambda t: (t, 0)),
                   pl.BlockSpec((_TILE, 8), lambda t: (t, 0))],
        out_shape=[jax.ShapeDtypeStruct((_TOT, 8), F32),
                   jax.ShapeDtypeStruct((_TOT, 8), F32)],
    )(*tab_args, *conv_args, *cls_args)

    return (z_out[:, 0], c_out)
